# Initial kernel scaffold; baseline (speedup 1.0000x reference)
import jax
import jax.numpy as jnp
from jax.experimental import pallas as pl


def kernel(img, w0, b0, alpha0, gamma0, beta0, w1, b1, alpha1, gamma1, beta1, w2, b2, alpha2, gamma2, beta2, w3, b3, alpha3, gamma3, beta3):
    raise NotImplementedError("write your pallas kernel here")


# speedup vs baseline: 2.0650x; 2.0650x over previous
import functools

import jax
import jax.numpy as jnp
import numpy as np
from jax.experimental import pallas as pl
from jax.experimental.pallas import tpu as pltpu

_EPS = 1e-5


def _pixel_shuffle_3d(x, scale):
    B, C, D, H, W = x.shape
    n_out = C // scale ** 3
    x = x.reshape(B, n_out, scale, scale, scale, D, H, W)
    x = jnp.transpose(x, (0, 1, 5, 2, 6, 3, 7, 4))
    return x.reshape(B, n_out, D * scale, H * scale, W * scale)


@functools.lru_cache(maxsize=None)
def _hw_masks_np(D, H, W):
    r = np.arange(D * H * W)
    h = (r // W) % H
    w = r % W
    m = np.zeros((9, D * H * W), np.float32)
    j = 0
    for oh in (-1, 0, 1):
        for ow in (-1, 0, 1):
            valid = ((h + oh >= 0) & (h + oh < H) &
                     (w + ow >= 0) & (w + ow < W))
            m[j] = valid.astype(np.float32)
            j += 1
    return m


def _roll_lanes(x, k):
    L = x.shape[-1]
    k %= L
    if k == 0:
        return x
    return jnp.concatenate([x[:, k:], x[:, :k]], axis=1)


def _shift_stack(x, m_ref, W):
    chunks = []
    j = 0
    for oh in (-1, 0, 1):
        for ow in (-1, 0, 1):
            xr = _roll_lanes(x, oh * W + ow)
            if not (oh == 0 and ow == 0):
                xr = xr * m_ref[j]
            chunks.append(xr)
            j += 1
    return jnp.concatenate(chunks, axis=0)


def _conv_core(x, wg_ref, b_ref, alpha, m_ref, W, HW, add_identity):
    S = _shift_stack(x, m_ref, W)
    p_lo = jnp.dot(wg_ref[0], S, preferred_element_type=jnp.float32)
    p_mid = jnp.dot(wg_ref[1], S, preferred_element_type=jnp.float32)
    p_hi = jnp.dot(wg_ref[2], S, preferred_element_type=jnp.float32)
    cout, L = p_mid.shape
    z = jnp.zeros((cout, HW), jnp.float32)
    acc = p_mid
    acc = acc + jnp.concatenate([z, p_lo[:, :L - HW]], axis=1)
    acc = acc + jnp.concatenate([p_hi[:, HW:], z], axis=1)
    c = acc + b_ref[...]
    y = jnp.where(c > 0, c, alpha * c)
    if add_identity:
        y = y + c
    return y


def _c0_kernel(alpha_ref, x_ref, wg_ref, b_ref, m_ref,
               a_ref, sum_ref, ssq_ref, *, W, HW):
    y = _conv_core(x_ref[0], wg_ref, b_ref, alpha_ref[0], m_ref, W, HW, True)
    a_ref[0] = y
    sum_ref[0] = jnp.sum(y, axis=1, keepdims=True)
    ssq_ref[0] = jnp.sum(y * y, axis=1, keepdims=True)


def _fused_kernel(alpha_ref, scale_ref, shift_ref, a_prev_ref, res_ref,
                  wg_ref, b_ref, m_ref,
                  cur_ref, a_ref, sum_ref, ssq_ref, *, W, HW, has_res):
    cur = a_prev_ref[0] * scale_ref[...] + shift_ref[...]
    if has_res:
        cur = cur + res_ref[0]
    cur_ref[0] = cur
    y = _conv_core(cur.astype(jnp.bfloat16), wg_ref, b_ref, alpha_ref[0],
                   m_ref, W, HW, False)
    a_ref[0] = y
    sum_ref[0] = jnp.sum(y, axis=1, keepdims=True)
    ssq_ref[0] = jnp.sum(y * y, axis=1, keepdims=True)


def _final_kernel(scale_ref, shift_ref, a_ref, res_ref, out_ref):
    out_ref[0] = a_ref[0] * scale_ref[...] + shift_ref[...] + res_ref[0]


def _prep_w(w, dtype):
    _, cout, cin = w.shape
    return (w.reshape(3, 9, cout, cin).transpose(0, 2, 1, 3)
            .reshape(3, cout, 9 * cin).astype(dtype))


def _bn_scale_shift(s1, s2, gamma, beta, n, add_self):
    s1 = jnp.sum(s1, axis=0).reshape(-1)
    s2 = jnp.sum(s2, axis=0).reshape(-1)
    mean = s1 / n
    var = jnp.maximum(s2 / n - mean * mean, 0.0)
    inv = gamma * jax.lax.rsqrt(var + _EPS)
    shift = beta - mean * inv
    scale = inv + (1.0 if add_self else 0.0)
    C = scale.shape[0]
    return scale.reshape(C, 1), shift.reshape(C, 1)


def _cparams():
    return pltpu.CompilerParams(
        dimension_semantics=("parallel",),
        vmem_limit_bytes=48 * 1024 * 1024)


def _conv0_call(alpha, x_flat, w, b, m32, *, W, HW):
    B, cin, L = x_flat.shape
    cout = w.shape[1]
    wg = _prep_w(w, jnp.float32)
    out_shape = (
        jax.ShapeDtypeStruct((B, cout, L), jnp.float32),
        jax.ShapeDtypeStruct((B, cout, 1), jnp.float32),
        jax.ShapeDtypeStruct((B, cout, 1), jnp.float32),
    )
    return pl.pallas_call(
        functools.partial(_c0_kernel, W=W, HW=HW),
        out_shape=out_shape,
        grid=(B,),
        in_specs=[
            pl.BlockSpec(memory_space=pltpu.MemorySpace.SMEM),
            pl.BlockSpec((1, cin, L), lambda b: (b, 0, 0)),
            pl.BlockSpec((3, cout, 9 * cin), lambda b: (0, 0, 0)),
            pl.BlockSpec((cout, 1), lambda b: (0, 0)),
            pl.BlockSpec((9, 1, L), lambda b: (0, 0, 0)),
        ],
        out_specs=(
            pl.BlockSpec((1, cout, L), lambda b: (b, 0, 0)),
            pl.BlockSpec((1, cout, 1), lambda b: (b, 0, 0)),
            pl.BlockSpec((1, cout, 1), lambda b: (b, 0, 0)),
        ),
        compiler_params=_cparams(),
    )(alpha, x_flat, wg, b.reshape(cout, 1), m32)


def _fused_call(alpha, scale, shift, a_prev, res, w, b, m16, *, W, HW):
    B, cin, L = a_prev.shape
    cout = w.shape[1]
    wg = _prep_w(w, jnp.bfloat16)
    has_res = res is not None
    bspec = pl.BlockSpec((1, cin, L), lambda b: (b, 0, 0))
    cspec = pl.BlockSpec((cout, 1), lambda b: (0, 0))
    in_specs = [
        pl.BlockSpec(memory_space=pltpu.MemorySpace.SMEM),
        cspec, cspec,
        bspec,
    ]
    args = [alpha, scale, shift, a_prev]
    if has_res:
        in_specs.append(bspec)
        args.append(res)
    in_specs += [
        pl.BlockSpec((3, cout, 9 * cin), lambda b: (0, 0, 0)),
        cspec,
        pl.BlockSpec((9, 1, L), lambda b: (0, 0, 0)),
    ]
    args += [wg, b.reshape(cout, 1), m16]

    def body(alpha_ref, scale_ref, shift_ref, a_prev_ref, *rest):
        if has_res:
            res_ref = rest[0]
            rest = rest[1:]
        else:
            res_ref = None
        wg_ref, b_ref, m_ref, cur_ref, a_ref, sum_ref, ssq_ref = rest
        _fused_kernel(alpha_ref, scale_ref, shift_ref, a_prev_ref, res_ref,
                      wg_ref, b_ref, m_ref, cur_ref, a_ref, sum_ref, ssq_ref,
                      W=W, HW=HW, has_res=has_res)

    out_shape = (
        jax.ShapeDtypeStruct((B, cout, L), jnp.float32),
        jax.ShapeDtypeStruct((B, cout, L), jnp.float32),
        jax.ShapeDtypeStruct((B, cout, 1), jnp.float32),
        jax.ShapeDtypeStruct((B, cout, 1), jnp.float32),
    )
    return pl.pallas_call(
        body,
        out_shape=out_shape,
        grid=(B,),
        in_specs=in_specs,
        out_specs=(
            pl.BlockSpec((1, cout, L), lambda b: (b, 0, 0)),
            pl.BlockSpec((1, cout, L), lambda b: (b, 0, 0)),
            pl.BlockSpec((1, cout, 1), lambda b: (b, 0, 0)),
            pl.BlockSpec((1, cout, 1), lambda b: (b, 0, 0)),
        ),
        compiler_params=_cparams(),
    )(*args)


def _final_call(scale, shift, a, res):
    B, cout, L = a.shape
    bspec = pl.BlockSpec((1, cout, L), lambda b: (b, 0, 0))
    cspec = pl.BlockSpec((cout, 1), lambda b: (0, 0))
    return pl.pallas_call(
        _final_kernel,
        out_shape=jax.ShapeDtypeStruct((B, cout, L), jnp.float32),
        grid=(B,),
        in_specs=[cspec, cspec, bspec, bspec],
        out_specs=bspec,
        compiler_params=_cparams(),
    )(scale, shift, a, res)


def kernel(img,
           w0, b0, alpha0, gamma0, beta0,
           w1, b1, alpha1, gamma1, beta1,
           w2, b2, alpha2, gamma2, beta2,
           w3, b3, alpha3, gamma3, beta3):
    x = _pixel_shuffle_3d(img, 2)
    B, C0, D, H, W = x.shape
    L = D * H * W
    HW = H * W
    x_flat = x.reshape(B, C0, L)
    n = B * L

    m32 = jnp.asarray(_hw_masks_np(D, H, W)).reshape(9, 1, L)
    m16 = m32.astype(jnp.bfloat16)

    a, s1, s2 = _conv0_call(alpha0, x_flat, w0, b0, m32, W=W, HW=HW)
    sc, sh = _bn_scale_shift(s1, s2, gamma0, beta0, n, True)

    res = None
    for (w, b, alpha, gamma, beta) in (
            (w1, b1, alpha1, gamma1, beta1),
            (w2, b2, alpha2, gamma2, beta2),
            (w3, b3, alpha3, gamma3, beta3)):
        cur, a_new, s1, s2 = _fused_call(alpha, sc, sh, a, res, w, b, m16,
                                         W=W, HW=HW)
        sc, sh = _bn_scale_shift(s1, s2, gamma, beta, n, False)
        a, res = a_new, cur

    out = _final_call(sc, sh, a, res)
    cout = out.shape[1]
    return out.reshape(B, cout, D, H, W)


# speedup vs baseline: 2.1162x; 1.0248x over previous
import functools

import jax
import jax.numpy as jnp
import numpy as np
from jax.experimental import pallas as pl
from jax.experimental.pallas import tpu as pltpu

_EPS = 1e-5


def _pixel_shuffle_3d(x, scale):
    B, C, D, H, W = x.shape
    n_out = C // scale ** 3
    x = x.reshape(B, n_out, scale, scale, scale, D, H, W)
    x = jnp.transpose(x, (0, 1, 5, 2, 6, 3, 7, 4))
    return x.reshape(B, n_out, D * scale, H * scale, W * scale)


@functools.lru_cache(maxsize=None)
def _hw_masks_np(D, H, W):
    r = np.arange(D * H * W)
    h = (r // W) % H
    w = r % W
    m = np.zeros((9, D * H * W), np.float32)
    j = 0
    for oh in (-1, 0, 1):
        for ow in (-1, 0, 1):
            valid = ((h + oh >= 0) & (h + oh < H) &
                     (w + ow >= 0) & (w + ow < W))
            m[j] = valid.astype(np.float32)
            j += 1
    return m


def _roll_lanes(x, k):
    L = x.shape[-1]
    k %= L
    if k == 0:
        return x
    return jnp.concatenate([x[:, k:], x[:, :k]], axis=1)


def _shift_stack(x, m_ref, W):
    chunks = []
    j = 0
    for oh in (-1, 0, 1):
        for ow in (-1, 0, 1):
            xr = _roll_lanes(x, oh * W + ow)
            if not (oh == 0 and ow == 0):
                xr = xr * m_ref[j]
            chunks.append(xr)
            j += 1
    return jnp.concatenate(chunks, axis=0)


def _conv_core(x, wg_ref, b_ref, alpha, m_ref, W, HW, add_identity):
    S = _shift_stack(x, m_ref, W)
    p_lo = jnp.dot(wg_ref[0], S, preferred_element_type=jnp.float32)
    p_mid = jnp.dot(wg_ref[1], S, preferred_element_type=jnp.float32)
    p_hi = jnp.dot(wg_ref[2], S, preferred_element_type=jnp.float32)
    cout, L = p_mid.shape
    z = jnp.zeros((cout, HW), jnp.float32)
    acc = p_mid
    acc = acc + jnp.concatenate([z, p_lo[:, :L - HW]], axis=1)
    acc = acc + jnp.concatenate([p_hi[:, HW:], z], axis=1)
    c = acc + b_ref[...]
    y = jnp.where(c > 0, c, alpha * c)
    if add_identity:
        y = y + c
    return y


def _c0_kernel(alpha_ref, x_ref, wg_ref, b_ref, m_ref,
               a_ref, sum_ref, ssq_ref, *, W, HW):
    y = _conv_core(x_ref[0], wg_ref, b_ref, alpha_ref[0], m_ref, W, HW, True)
    a_ref[0] = y.astype(jnp.bfloat16)
    sum_ref[0] = jnp.sum(y, axis=1, keepdims=True)
    ssq_ref[0] = jnp.sum(y * y, axis=1, keepdims=True)


def _fused_kernel(alpha_ref, scale_ref, shift_ref, a_prev_ref, res_ref,
                  wg_ref, b_ref, m_ref,
                  cur_ref, a_ref, sum_ref, ssq_ref, *, W, HW, has_res):
    cur = a_prev_ref[0] * scale_ref[...] + shift_ref[...]
    if has_res:
        cur = cur + res_ref[0]
    x16 = cur.astype(jnp.bfloat16)
    cur_ref[0] = x16
    y = _conv_core(x16, wg_ref, b_ref, alpha_ref[0], m_ref, W, HW, False)
    a_ref[0] = y.astype(jnp.bfloat16)
    sum_ref[0] = jnp.sum(y, axis=1, keepdims=True)
    ssq_ref[0] = jnp.sum(y * y, axis=1, keepdims=True)


def _final_kernel(scale_ref, shift_ref, a_ref, res_ref, out_ref):
    out_ref[0] = a_ref[0] * scale_ref[...] + shift_ref[...] + res_ref[0]


def _prep_w(w, dtype):
    _, cout, cin = w.shape
    return (w.reshape(3, 9, cout, cin).transpose(0, 2, 1, 3)
            .reshape(3, cout, 9 * cin).astype(dtype))


def _bn_scale_shift(s1, s2, gamma, beta, n, add_self):
    s1 = jnp.sum(s1, axis=0).reshape(-1)
    s2 = jnp.sum(s2, axis=0).reshape(-1)
    mean = s1 / n
    var = jnp.maximum(s2 / n - mean * mean, 0.0)
    inv = gamma * jax.lax.rsqrt(var + _EPS)
    shift = beta - mean * inv
    scale = inv + (1.0 if add_self else 0.0)
    C = scale.shape[0]
    return scale.reshape(C, 1), shift.reshape(C, 1)


def _cparams():
    return pltpu.CompilerParams(
        dimension_semantics=("parallel",),
        vmem_limit_bytes=48 * 1024 * 1024)


def _conv0_call(alpha, x_flat, w, b, m32, *, W, HW):
    B, cin, L = x_flat.shape
    cout = w.shape[1]
    wg = _prep_w(w, jnp.float32)
    out_shape = (
        jax.ShapeDtypeStruct((B, cout, L), jnp.bfloat16),
        jax.ShapeDtypeStruct((B, cout, 1), jnp.float32),
        jax.ShapeDtypeStruct((B, cout, 1), jnp.float32),
    )
    return pl.pallas_call(
        functools.partial(_c0_kernel, W=W, HW=HW),
        out_shape=out_shape,
        grid=(B,),
        in_specs=[
            pl.BlockSpec(memory_space=pltpu.MemorySpace.SMEM),
            pl.BlockSpec((1, cin, L), lambda b: (b, 0, 0)),
            pl.BlockSpec((3, cout, 9 * cin), lambda b: (0, 0, 0)),
            pl.BlockSpec((cout, 1), lambda b: (0, 0)),
            pl.BlockSpec((9, 1, L), lambda b: (0, 0, 0)),
        ],
        out_specs=(
            pl.BlockSpec((1, cout, L), lambda b: (b, 0, 0)),
            pl.BlockSpec((1, cout, 1), lambda b: (b, 0, 0)),
            pl.BlockSpec((1, cout, 1), lambda b: (b, 0, 0)),
        ),
        compiler_params=_cparams(),
    )(alpha, x_flat, wg, b.reshape(cout, 1), m32)


def _fused_call(alpha, scale, shift, a_prev, res, w, b, m16, *, W, HW):
    B, cin, L = a_prev.shape
    cout = w.shape[1]
    wg = _prep_w(w, jnp.bfloat16)
    has_res = res is not None
    bspec = pl.BlockSpec((1, cin, L), lambda b: (b, 0, 0))
    cspec = pl.BlockSpec((cout, 1), lambda b: (0, 0))
    in_specs = [
        pl.BlockSpec(memory_space=pltpu.MemorySpace.SMEM),
        cspec, cspec,
        bspec,
    ]
    args = [alpha, scale, shift, a_prev]
    if has_res:
        in_specs.append(bspec)
        args.append(res)
    in_specs += [
        pl.BlockSpec((3, cout, 9 * cin), lambda b: (0, 0, 0)),
        cspec,
        pl.BlockSpec((9, 1, L), lambda b: (0, 0, 0)),
    ]
    args += [wg, b.reshape(cout, 1), m16]

    def body(alpha_ref, scale_ref, shift_ref, a_prev_ref, *rest):
        if has_res:
            res_ref = rest[0]
            rest = rest[1:]
        else:
            res_ref = None
        wg_ref, b_ref, m_ref, cur_ref, a_ref, sum_ref, ssq_ref = rest
        _fused_kernel(alpha_ref, scale_ref, shift_ref, a_prev_ref, res_ref,
                      wg_ref, b_ref, m_ref, cur_ref, a_ref, sum_ref, ssq_ref,
                      W=W, HW=HW, has_res=has_res)

    out_shape = (
        jax.ShapeDtypeStruct((B, cout, L), jnp.bfloat16),
        jax.ShapeDtypeStruct((B, cout, L), jnp.bfloat16),
        jax.ShapeDtypeStruct((B, cout, 1), jnp.float32),
        jax.ShapeDtypeStruct((B, cout, 1), jnp.float32),
    )
    return pl.pallas_call(
        body,
        out_shape=out_shape,
        grid=(B,),
        in_specs=in_specs,
        out_specs=(
            pl.BlockSpec((1, cout, L), lambda b: (b, 0, 0)),
            pl.BlockSpec((1, cout, L), lambda b: (b, 0, 0)),
            pl.BlockSpec((1, cout, 1), lambda b: (b, 0, 0)),
            pl.BlockSpec((1, cout, 1), lambda b: (b, 0, 0)),
        ),
        compiler_params=_cparams(),
    )(*args)


def _final_call(scale, shift, a, res):
    B, cout, L = a.shape
    bspec = pl.BlockSpec((1, cout, L), lambda b: (b, 0, 0))
    cspec = pl.BlockSpec((cout, 1), lambda b: (0, 0))
    return pl.pallas_call(
        _final_kernel,
        out_shape=jax.ShapeDtypeStruct((B, cout, L), jnp.float32),
        grid=(B,),
        in_specs=[cspec, cspec, bspec, bspec],
        out_specs=bspec,
        compiler_params=_cparams(),
    )(scale, shift, a, res)


def kernel(img,
           w0, b0, alpha0, gamma0, beta0,
           w1, b1, alpha1, gamma1, beta1,
           w2, b2, alpha2, gamma2, beta2,
           w3, b3, alpha3, gamma3, beta3):
    x = _pixel_shuffle_3d(img, 2)
    B, C0, D, H, W = x.shape
    L = D * H * W
    HW = H * W
    x_flat = x.reshape(B, C0, L)
    n = B * L

    m32 = jnp.asarray(_hw_masks_np(D, H, W)).reshape(9, 1, L)
    m16 = m32.astype(jnp.bfloat16)

    a, s1, s2 = _conv0_call(alpha0, x_flat, w0, b0, m32, W=W, HW=HW)
    sc, sh = _bn_scale_shift(s1, s2, gamma0, beta0, n, True)

    res = None
    for (w, b, alpha, gamma, beta) in (
            (w1, b1, alpha1, gamma1, beta1),
            (w2, b2, alpha2, gamma2, beta2),
            (w3, b3, alpha3, gamma3, beta3)):
        cur, a_new, s1, s2 = _fused_call(alpha, sc, sh, a, res, w, b, m16,
                                         W=W, HW=HW)
        sc, sh = _bn_scale_shift(s1, s2, gamma, beta, n, False)
        a, res = a_new, cur

    out = _final_call(sc, sh, a, res)
    cout = out.shape[1]
    return out.reshape(B, cout, D, H, W)


# speedup vs baseline: 2.4804x; 1.1721x over previous
import functools

import jax
import jax.numpy as jnp
import numpy as np
from jax.experimental import pallas as pl
from jax.experimental.pallas import tpu as pltpu

_EPS = 1e-5


def _pixel_shuffle_3d(x, scale):
    B, C, D, H, W = x.shape
    n_out = C // scale ** 3
    x = x.reshape(B, n_out, scale, scale, scale, D, H, W)
    x = jnp.transpose(x, (0, 1, 5, 2, 6, 3, 7, 4))
    return x.reshape(B, n_out, D * scale, H * scale, W * scale)


@functools.lru_cache(maxsize=None)
def _hw_masks_np(D, H, W):
    r = np.arange(D * H * W)
    h = (r // W) % H
    w = r % W
    m = np.zeros((9, D * H * W), np.float32)
    j = 0
    for oh in (-1, 0, 1):
        for ow in (-1, 0, 1):
            valid = ((h + oh >= 0) & (h + oh < H) &
                     (w + ow >= 0) & (w + ow < W))
            m[j] = valid.astype(np.float32)
            j += 1
    return m


def _roll_lanes(x, k):
    L = x.shape[-1]
    k %= L
    if k == 0:
        return x
    return jnp.concatenate([x[:, k:], x[:, :k]], axis=1)


def _shift_stack(x, m_ref, W):
    chunks = []
    j = 0
    for oh in (-1, 0, 1):
        for ow in (-1, 0, 1):
            xr = _roll_lanes(x, oh * W + ow)
            if not (oh == 0 and ow == 0):
                xr = xr * m_ref[j]
            chunks.append(xr)
            j += 1
    return jnp.concatenate(chunks, axis=0)


def _conv_core(x, wg_ref, b_ref, alpha, m_ref, W, HW, add_identity):
    S = _shift_stack(x, m_ref, W)
    p_lo = jnp.dot(wg_ref[0], S, preferred_element_type=jnp.float32)
    p_mid = jnp.dot(wg_ref[1], S, preferred_element_type=jnp.float32)
    p_hi = jnp.dot(wg_ref[2], S, preferred_element_type=jnp.float32)
    cout, L = p_mid.shape
    z = jnp.zeros((cout, HW), jnp.float32)
    acc = p_mid
    acc = acc + jnp.concatenate([z, p_lo[:, :L - HW]], axis=1)
    acc = acc + jnp.concatenate([p_hi[:, HW:], z], axis=1)
    c = acc + b_ref[...]
    y = jnp.where(c > 0, c, alpha * c)
    if add_identity:
        y = y + c
    return y


def _c0_kernel(alpha_ref, x_ref, wg_ref, b_ref, m_ref,
               a_ref, sum_ref, ssq_ref, *, W, HW, nb):
    for i in range(nb):
        y = _conv_core(x_ref[i], wg_ref, b_ref, alpha_ref[0], m_ref,
                       W, HW, True)
        a_ref[i] = y.astype(jnp.bfloat16)
        sum_ref[i] = jnp.sum(y, axis=1, keepdims=True)
        ssq_ref[i] = jnp.sum(y * y, axis=1, keepdims=True)


def _fused_kernel(alpha_ref, scale_ref, shift_ref, a_prev_ref, res_ref,
                  wg_ref, b_ref, m_ref,
                  cur_ref, a_ref, sum_ref, ssq_ref, *, W, HW, has_res, nb):
    for i in range(nb):
        cur = a_prev_ref[i] * scale_ref[...] + shift_ref[...]
        if has_res:
            cur = cur + res_ref[i]
        x16 = cur.astype(jnp.bfloat16)
        cur_ref[i] = x16
        y = _conv_core(x16, wg_ref, b_ref, alpha_ref[0], m_ref, W, HW, False)
        a_ref[i] = y.astype(jnp.bfloat16)
        sum_ref[i] = jnp.sum(y, axis=1, keepdims=True)
        ssq_ref[i] = jnp.sum(y * y, axis=1, keepdims=True)


def _final_kernel(scale_ref, shift_ref, a_ref, res_ref, out_ref):
    out_ref[...] = (a_ref[...] * scale_ref[...] + shift_ref[...]
                    + res_ref[...])


def _prep_w(w, dtype):
    _, cout, cin = w.shape
    return (w.reshape(3, 9, cout, cin).transpose(0, 2, 1, 3)
            .reshape(3, cout, 9 * cin).astype(dtype))


def _bn_scale_shift(s1, s2, gamma, beta, n, add_self):
    s1 = jnp.sum(s1, axis=0).reshape(-1)
    s2 = jnp.sum(s2, axis=0).reshape(-1)
    mean = s1 / n
    var = jnp.maximum(s2 / n - mean * mean, 0.0)
    inv = gamma * jax.lax.rsqrt(var + _EPS)
    shift = beta - mean * inv
    scale = inv + (1.0 if add_self else 0.0)
    C = scale.shape[0]
    return scale.reshape(C, 1), shift.reshape(C, 1)


def _cparams():
    return pltpu.CompilerParams(
        dimension_semantics=("parallel",),
        vmem_limit_bytes=48 * 1024 * 1024)


def _conv0_call(alpha, x_flat, w, b, m32, *, W, HW, nb):
    B, cin, L = x_flat.shape
    cout = w.shape[1]
    wg = _prep_w(w, jnp.float32)
    out_shape = (
        jax.ShapeDtypeStruct((B, cout, L), jnp.bfloat16),
        jax.ShapeDtypeStruct((B, cout, 1), jnp.float32),
        jax.ShapeDtypeStruct((B, cout, 1), jnp.float32),
    )
    return pl.pallas_call(
        functools.partial(_c0_kernel, W=W, HW=HW, nb=nb),
        out_shape=out_shape,
        grid=(B // nb,),
        in_specs=[
            pl.BlockSpec(memory_space=pltpu.MemorySpace.SMEM),
            pl.BlockSpec((nb, cin, L), lambda b: (b, 0, 0)),
            pl.BlockSpec((3, cout, 9 * cin), lambda b: (0, 0, 0)),
            pl.BlockSpec((cout, 1), lambda b: (0, 0)),
            pl.BlockSpec((9, 1, L), lambda b: (0, 0, 0)),
        ],
        out_specs=(
            pl.BlockSpec((nb, cout, L), lambda b: (b, 0, 0)),
            pl.BlockSpec((nb, cout, 1), lambda b: (b, 0, 0)),
            pl.BlockSpec((nb, cout, 1), lambda b: (b, 0, 0)),
        ),
        compiler_params=_cparams(),
    )(alpha, x_flat, wg, b.reshape(cout, 1), m32)


def _fused_call(alpha, scale, shift, a_prev, res, w, b, m16, *, W, HW, nb):
    B, cin, L = a_prev.shape
    cout = w.shape[1]
    wg = _prep_w(w, jnp.bfloat16)
    has_res = res is not None
    bspec = pl.BlockSpec((nb, cin, L), lambda b: (b, 0, 0))
    cspec = pl.BlockSpec((cout, 1), lambda b: (0, 0))
    in_specs = [
        pl.BlockSpec(memory_space=pltpu.MemorySpace.SMEM),
        cspec, cspec,
        bspec,
    ]
    args = [alpha, scale, shift, a_prev]
    if has_res:
        in_specs.append(bspec)
        args.append(res)
    in_specs += [
        pl.BlockSpec((3, cout, 9 * cin), lambda b: (0, 0, 0)),
        cspec,
        pl.BlockSpec((9, 1, L), lambda b: (0, 0, 0)),
    ]
    args += [wg, b.reshape(cout, 1), m16]

    def body(alpha_ref, scale_ref, shift_ref, a_prev_ref, *rest):
        if has_res:
            res_ref = rest[0]
            rest = rest[1:]
        else:
            res_ref = None
        wg_ref, b_ref, m_ref, cur_ref, a_ref, sum_ref, ssq_ref = rest
        _fused_kernel(alpha_ref, scale_ref, shift_ref, a_prev_ref, res_ref,
                      wg_ref, b_ref, m_ref, cur_ref, a_ref, sum_ref, ssq_ref,
                      W=W, HW=HW, has_res=has_res, nb=nb)

    out_shape = (
        jax.ShapeDtypeStruct((B, cout, L), jnp.bfloat16),
        jax.ShapeDtypeStruct((B, cout, L), jnp.bfloat16),
        jax.ShapeDtypeStruct((B, cout, 1), jnp.float32),
        jax.ShapeDtypeStruct((B, cout, 1), jnp.float32),
    )
    return pl.pallas_call(
        body,
        out_shape=out_shape,
        grid=(B // nb,),
        in_specs=in_specs,
        out_specs=(
            pl.BlockSpec((nb, cout, L), lambda b: (b, 0, 0)),
            pl.BlockSpec((nb, cout, L), lambda b: (b, 0, 0)),
            pl.BlockSpec((nb, cout, 1), lambda b: (b, 0, 0)),
            pl.BlockSpec((nb, cout, 1), lambda b: (b, 0, 0)),
        ),
        compiler_params=_cparams(),
    )(*args)


def _final_call(scale, shift, a, res, *, nb):
    B, cout, L = a.shape
    bspec = pl.BlockSpec((nb, cout, L), lambda b: (b, 0, 0))
    cspec = pl.BlockSpec((cout, 1), lambda b: (0, 0))
    return pl.pallas_call(
        _final_kernel,
        out_shape=jax.ShapeDtypeStruct((B, cout, L), jnp.float32),
        grid=(B // nb,),
        in_specs=[cspec, cspec, bspec, bspec],
        out_specs=bspec,
        compiler_params=_cparams(),
    )(scale, shift, a, res)


def kernel(img,
           w0, b0, alpha0, gamma0, beta0,
           w1, b1, alpha1, gamma1, beta1,
           w2, b2, alpha2, gamma2, beta2,
           w3, b3, alpha3, gamma3, beta3):
    x = _pixel_shuffle_3d(img, 2)
    B, C0, D, H, W = x.shape
    L = D * H * W
    HW = H * W
    x_flat = x.reshape(B, C0, L)
    n = B * L

    m32 = jnp.asarray(_hw_masks_np(D, H, W)).reshape(9, 1, L)
    m16 = m32.astype(jnp.bfloat16)

    nbc = 2 if B % 2 == 0 else 1
    nbf = 4 if B % 4 == 0 else nbc
    a, s1, s2 = _conv0_call(alpha0, x_flat, w0, b0, m32, W=W, HW=HW, nb=nbc)
    sc, sh = _bn_scale_shift(s1, s2, gamma0, beta0, n, True)

    res = None
    for (w, b, alpha, gamma, beta) in (
            (w1, b1, alpha1, gamma1, beta1),
            (w2, b2, alpha2, gamma2, beta2),
            (w3, b3, alpha3, gamma3, beta3)):
        cur, a_new, s1, s2 = _fused_call(alpha, sc, sh, a, res, w, b, m16,
                                         W=W, HW=HW, nb=nbc)
        sc, sh = _bn_scale_shift(s1, s2, gamma, beta, n, False)
        a, res = a_new, cur

    out = _final_call(sc, sh, a, res, nb=nbf)
    cout = out.shape[1]
    return out.reshape(B, cout, D, H, W)


# speedup vs baseline: 2.6343x; 1.0621x over previous
import functools

import jax
import jax.numpy as jnp
import numpy as np
from jax.experimental import pallas as pl
from jax.experimental.pallas import tpu as pltpu

_EPS = 1e-5


def _pixel_shuffle_3d(x, scale):
    B, C, D, H, W = x.shape
    n_out = C // scale ** 3
    x = x.reshape(B, n_out, scale, scale, scale, D, H, W)
    x = jnp.transpose(x, (0, 1, 5, 2, 6, 3, 7, 4))
    return x.reshape(B, n_out, D * scale, H * scale, W * scale)


@functools.lru_cache(maxsize=None)
def _hw_masks_np(D, H, W):
    r = np.arange(D * H * W)
    h = (r // W) % H
    w = r % W
    m = np.zeros((9, D * H * W), np.float32)
    j = 0
    for oh in (-1, 0, 1):
        for ow in (-1, 0, 1):
            valid = ((h + oh >= 0) & (h + oh < H) &
                     (w + ow >= 0) & (w + ow < W))
            m[j] = valid.astype(np.float32)
            j += 1
    return m


def _roll_lanes(x, k):
    L = x.shape[-1]
    k %= L
    if k == 0:
        return x
    return jnp.concatenate([x[:, k:], x[:, :k]], axis=1)


def _shift_stack(x, m_ref, W):
    chunks = []
    j = 0
    for oh in (-1, 0, 1):
        for ow in (-1, 0, 1):
            xr = _roll_lanes(x, oh * W + ow)
            if not (oh == 0 and ow == 0):
                xr = xr * m_ref[j]
            chunks.append(xr)
            j += 1
    return jnp.concatenate(chunks, axis=0)


def _conv_core(x, wg_ref, b_ref, alpha, m_ref, W, HW, add_identity):
    S = _shift_stack(x, m_ref, W)
    p_lo = jnp.dot(wg_ref[0], S, preferred_element_type=jnp.float32)
    p_mid = jnp.dot(wg_ref[1], S, preferred_element_type=jnp.float32)
    p_hi = jnp.dot(wg_ref[2], S, preferred_element_type=jnp.float32)
    cout, L = p_mid.shape
    z = jnp.zeros((cout, HW), jnp.float32)
    acc = p_mid
    acc = acc + jnp.concatenate([z, p_lo[:, :L - HW]], axis=1)
    acc = acc + jnp.concatenate([p_hi[:, HW:], z], axis=1)
    c = acc + b_ref[...]
    y = jnp.where(c > 0, c, alpha * c)
    if add_identity:
        y = y + c
    return y


def _c0_kernel(alpha_ref, x_ref, wg_ref, b_ref, m_ref,
               a_ref, sum_ref, ssq_ref, *, W, HW, nb):
    for i in range(nb):
        y = _conv_core(x_ref[i], wg_ref, b_ref, alpha_ref[0], m_ref,
                       W, HW, True)
        a_ref[i] = y.astype(jnp.bfloat16)
        sum_ref[i] = jnp.sum(y, axis=1, keepdims=True)
        ssq_ref[i] = jnp.sum(y * y, axis=1, keepdims=True)


def _fused_kernel(alpha_ref, scale_ref, shift_ref, a_prev_ref, res_ref,
                  wg_ref, b_ref, m_ref,
                  cur_ref, a_ref, sum_ref, ssq_ref, *, W, HW, has_res, nb):
    for i in range(nb):
        cur = a_prev_ref[i] * scale_ref[...] + shift_ref[...]
        if has_res:
            cur = cur + res_ref[i]
        x16 = cur.astype(jnp.bfloat16)
        cur_ref[i] = x16
        y = _conv_core(x16, wg_ref, b_ref, alpha_ref[0], m_ref, W, HW, False)
        a_ref[i] = y.astype(jnp.bfloat16)
        sum_ref[i] = jnp.sum(y, axis=1, keepdims=True)
        ssq_ref[i] = jnp.sum(y * y, axis=1, keepdims=True)


def _final_kernel(scale_ref, shift_ref, a_ref, res_ref, out_ref):
    out_ref[...] = (a_ref[...] * scale_ref[...] + shift_ref[...]
                    + res_ref[...])


def _prep_w(w, dtype):
    _, cout, cin = w.shape
    return (w.reshape(3, 9, cout, cin).transpose(0, 2, 1, 3)
            .reshape(3, cout, 9 * cin).astype(dtype))


def _bn_scale_shift(s1, s2, gamma, beta, n, add_self):
    s1 = jnp.sum(s1, axis=0).reshape(-1)
    s2 = jnp.sum(s2, axis=0).reshape(-1)
    mean = s1 / n
    var = jnp.maximum(s2 / n - mean * mean, 0.0)
    inv = gamma * jax.lax.rsqrt(var + _EPS)
    shift = beta - mean * inv
    scale = inv + (1.0 if add_self else 0.0)
    C = scale.shape[0]
    return scale.reshape(C, 1), shift.reshape(C, 1)


def _cparams():
    return pltpu.CompilerParams(
        dimension_semantics=("parallel",),
        vmem_limit_bytes=48 * 1024 * 1024)


def _conv0_call(alpha, x_flat, w, b, m32, *, W, HW, nb):
    B, cin, L = x_flat.shape
    cout = w.shape[1]
    wg = _prep_w(w, jnp.float32)
    out_shape = (
        jax.ShapeDtypeStruct((B, cout, L), jnp.bfloat16),
        jax.ShapeDtypeStruct((B, cout, 1), jnp.float32),
        jax.ShapeDtypeStruct((B, cout, 1), jnp.float32),
    )
    return pl.pallas_call(
        functools.partial(_c0_kernel, W=W, HW=HW, nb=nb),
        out_shape=out_shape,
        grid=(B // nb,),
        in_specs=[
            pl.BlockSpec(memory_space=pltpu.MemorySpace.SMEM),
            pl.BlockSpec((nb, cin, L), lambda b: (b, 0, 0)),
            pl.BlockSpec((3, cout, 9 * cin), lambda b: (0, 0, 0)),
            pl.BlockSpec((cout, 1), lambda b: (0, 0)),
            pl.BlockSpec((9, 1, L), lambda b: (0, 0, 0)),
        ],
        out_specs=(
            pl.BlockSpec((nb, cout, L), lambda b: (b, 0, 0)),
            pl.BlockSpec((nb, cout, 1), lambda b: (b, 0, 0)),
            pl.BlockSpec((nb, cout, 1), lambda b: (b, 0, 0)),
        ),
        compiler_params=_cparams(),
    )(alpha, x_flat, wg, b.reshape(cout, 1), m32)


def _fused_call(alpha, scale, shift, a_prev, res, w, b, m16, *, W, HW, nb):
    B, cin, L = a_prev.shape
    cout = w.shape[1]
    wg = _prep_w(w, jnp.bfloat16)
    has_res = res is not None
    bspec = pl.BlockSpec((nb, cin, L), lambda b: (b, 0, 0))
    cspec = pl.BlockSpec((cout, 1), lambda b: (0, 0))
    in_specs = [
        pl.BlockSpec(memory_space=pltpu.MemorySpace.SMEM),
        cspec, cspec,
        bspec,
    ]
    args = [alpha, scale, shift, a_prev]
    if has_res:
        in_specs.append(bspec)
        args.append(res)
    in_specs += [
        pl.BlockSpec((3, cout, 9 * cin), lambda b: (0, 0, 0)),
        cspec,
        pl.BlockSpec((9, 1, L), lambda b: (0, 0, 0)),
    ]
    args += [wg, b.reshape(cout, 1), m16]

    def body(alpha_ref, scale_ref, shift_ref, a_prev_ref, *rest):
        if has_res:
            res_ref = rest[0]
            rest = rest[1:]
        else:
            res_ref = None
        wg_ref, b_ref, m_ref, cur_ref, a_ref, sum_ref, ssq_ref = rest
        _fused_kernel(alpha_ref, scale_ref, shift_ref, a_prev_ref, res_ref,
                      wg_ref, b_ref, m_ref, cur_ref, a_ref, sum_ref, ssq_ref,
                      W=W, HW=HW, has_res=has_res, nb=nb)

    out_shape = (
        jax.ShapeDtypeStruct((B, cout, L), jnp.bfloat16),
        jax.ShapeDtypeStruct((B, cout, L), jnp.bfloat16),
        jax.ShapeDtypeStruct((B, cout, 1), jnp.float32),
        jax.ShapeDtypeStruct((B, cout, 1), jnp.float32),
    )
    return pl.pallas_call(
        body,
        out_shape=out_shape,
        grid=(B // nb,),
        in_specs=in_specs,
        out_specs=(
            pl.BlockSpec((nb, cout, L), lambda b: (b, 0, 0)),
            pl.BlockSpec((nb, cout, L), lambda b: (b, 0, 0)),
            pl.BlockSpec((nb, cout, 1), lambda b: (b, 0, 0)),
            pl.BlockSpec((nb, cout, 1), lambda b: (b, 0, 0)),
        ),
        compiler_params=_cparams(),
    )(*args)


def _final_call(scale, shift, a, res, *, nb):
    B, cout, L = a.shape
    bspec = pl.BlockSpec((nb, cout, L), lambda b: (b, 0, 0))
    cspec = pl.BlockSpec((cout, 1), lambda b: (0, 0))
    return pl.pallas_call(
        _final_kernel,
        out_shape=jax.ShapeDtypeStruct((B, cout, L), jnp.float32),
        grid=(B // nb,),
        in_specs=[cspec, cspec, bspec, bspec],
        out_specs=bspec,
        compiler_params=_cparams(),
    )(scale, shift, a, res)


def kernel(img,
           w0, b0, alpha0, gamma0, beta0,
           w1, b1, alpha1, gamma1, beta1,
           w2, b2, alpha2, gamma2, beta2,
           w3, b3, alpha3, gamma3, beta3):
    x = _pixel_shuffle_3d(img, 2)
    B, C0, D, H, W = x.shape
    L = D * H * W
    HW = H * W
    x_flat = x.reshape(B, C0, L)
    n = B * L

    m32 = jnp.asarray(_hw_masks_np(D, H, W)).reshape(9, 1, L)
    m16 = m32.astype(jnp.bfloat16)

    nbc = 4 if B % 4 == 0 else 1
    nbf = 8 if B % 8 == 0 else nbc
    a, s1, s2 = _conv0_call(alpha0, x_flat, w0, b0, m32, W=W, HW=HW, nb=nbc)
    sc, sh = _bn_scale_shift(s1, s2, gamma0, beta0, n, True)

    res = None
    for (w, b, alpha, gamma, beta) in (
            (w1, b1, alpha1, gamma1, beta1),
            (w2, b2, alpha2, gamma2, beta2),
            (w3, b3, alpha3, gamma3, beta3)):
        cur, a_new, s1, s2 = _fused_call(alpha, sc, sh, a, res, w, b, m16,
                                         W=W, HW=HW, nb=nbc)
        sc, sh = _bn_scale_shift(s1, s2, gamma, beta, n, False)
        a, res = a_new, cur

    out = _final_call(sc, sh, a, res, nb=nbf)
    cout = out.shape[1]
    return out.reshape(B, cout, D, H, W)
